# baseline (speedup 1.0000x reference)
import functools

import jax
import jax.numpy as jnp
from jax import lax
from jax.experimental import pallas as pl
from jax.experimental.pallas import tpu as pltpu
from jax.experimental.pallas import tpu_sc as plsc

NCORES = 2
NSUB = 16
CH = 128
BLK = 64


def _tc_feat_body(x_ref, w_ref, wa_ref, bh_ref, ba_ref, feat_ref, ac_ref):
    f = jnp.dot(x_ref[...], w_ref[...], preferred_element_type=jnp.float32)
    f = f + bh_ref[...]
    feat_ref[...] = f
    ac_ref[...] = (
        jnp.dot(f, wa_ref[...], preferred_element_type=jnp.float32) + ba_ref[...]
    )


def _tc_combine_body(a0_ref, a1_ref, s_ref, out_ref):
    s = s_ref[...]
    num = jnp.concatenate([a0_ref[...], a1_ref[...]], axis=1)
    out_ref[...] = jnp.where(s > 0.0, num / s, 0.0)


def _make_sc_kernel(npad, nsum, ntab, nch, hdim):
    mesh = plsc.VectorSubcoreMesh(
        core_axis_name="c", subcore_axis_name="s",
        num_cores=NCORES, num_subcores=NSUB,
    )
    rows_per_sub = npad // NSUB
    srows_per_sub = nsum // NSUB

    @functools.partial(
        pl.kernel,
        out_type=[
            jax.ShapeDtypeStruct(
                (NCORES, NSUB, rows_per_sub, hdim), jnp.float32),
            jax.ShapeDtypeStruct((NCORES * nsum,), jnp.float32),
        ],
        mesh=mesh,
        compiler_params=pltpu.CompilerParams(
            needs_layout_passes=False, use_tc_tiling_on_sc=False),
        scratch_types=[
            pltpu.VMEM((ntab,), jnp.float32),
            pltpu.VMEM((ntab,), jnp.float32),
            pltpu.VMEM((nch + 2, CH), jnp.int32),
            pltpu.VMEM((nch + 2, CH), jnp.int32),
            pltpu.VMEM((CH, hdim), jnp.float32),
            pltpu.VMEM((CH, hdim), jnp.float32),
            pltpu.VMEM((CH, hdim), jnp.float32),
            pltpu.VMEM((CH, hdim), jnp.float32),
            pltpu.VMEM((CH,), jnp.float32),
            pltpu.VMEM((CH,), jnp.float32),
            pltpu.VMEM((640,), jnp.float32),
            pltpu.SemaphoreType.DMA,
            pltpu.SemaphoreType.DMA,
            pltpu.SemaphoreType.DMA,
            pltpu.SemaphoreType.DMA,
            pltpu.SemaphoreType.DMA,
            pltpu.SemaphoreType.DMA,
            pltpu.VMEM_SHARED((npad, hdim), jnp.float32),
            pltpu.VMEM_SHARED((nsum,), jnp.float32),
        ],
    )
    def sc_kernel(feat2_hbm, a_hbm, c_hbm, src_hbm, dst_hbm,
                  acc_hbm, sums_hbm, a_v, c_v, src_v, dst_v,
                  inA, inB, outA, outB, pA, pB, st_v,
                  gsA, gsB, ssA, ssB, usA, usB, acc_s, sums_s):
        cid = lax.axis_index("c")
        sid = lax.axis_index("s")

        def g_desc(ci, buf, sem):
            return pltpu.make_async_copy(feat2_hbm.at[dst_v.at[ci]], buf, sem)

        def s_desc(ci, buf, sem):
            return pltpu.make_async_copy(buf, acc_s.at[src_v.at[ci]], sem)

        def u_desc(ci, pbuf, sem):
            return pltpu.make_async_copy(pbuf, sums_s.at[src_v.at[ci]], sem)

        pltpu.sync_copy(a_hbm, a_v)
        pltpu.sync_copy(c_hbm, c_v)
        pltpu.sync_copy(src_hbm.at[sid], src_v)
        pltpu.sync_copy(dst_hbm.at[sid], dst_v)

        cid16 = jnp.full((16,), cid, jnp.int32)

        def dxf(r, carry):
            for g in range(CH // 16):
                sl = pl.ds(g * 16, 16)
                dst_v[r, sl] = dst_v[r, sl] * 2 + cid16
            return carry

        lax.fori_loop(0, nch + 2, dxf, 0)

        g_desc(0, inA, gsA).start()
        g_desc(1, inB, gsB).start()

        zero16 = jnp.zeros((16,), jnp.float32)

        def zrow(i, carry):
            for v in range(hdim // 16):
                outA[i, pl.ds(v * 16, 16)] = zero16
            return carry

        lax.fori_loop(0, CH, zrow, 0)

        def zst(i, carry):
            st_v[pl.ds(i * 16, 16)] = zero16
            return carry

        lax.fori_loop(0, 640 // 16, zst, 0)

        row0 = sid * rows_per_sub
        nfull, rem = divmod(rows_per_sub, CH)
        for k in range(nfull):
            pltpu.sync_copy(outA, acc_s.at[pl.ds(row0 + k * CH, CH)])
        if rem:
            pltpu.sync_copy(outA.at[pl.ds(0, rem)],
                            acc_s.at[pl.ds(row0 + nfull * CH, rem)])
        srow0 = sid * srows_per_sub
        pltpu.sync_copy(st_v.at[pl.ds(0, srows_per_sub)],
                        sums_s.at[pl.ds(srow0, srows_per_sub)])

        plsc.subcore_barrier()

        def scale(ci, inb, outb, pbuf):
            def group(j, carry):
                base = j * 16
                s16 = src_v[ci, pl.ds(base, 16)]
                d2 = dst_v[ci, pl.ds(base, 16)]
                d16 = lax.shift_right_logical(d2 - cid16, 1)
                av = plsc.load_gather(a_v, [s16])
                cv = plsc.load_gather(c_v, [d16])
                x = av + cv
                p16 = jnp.exp(jnp.maximum(x, 0.2 * x))
                pbuf[pl.ds(base, 16)] = p16
                for l in range(16):
                    ps = jnp.full((16,), p16[l], jnp.float32)
                    for v in range(hdim // 16):
                        sl = pl.ds(v * 16, 16)
                        outb[base + l, sl] = inb[base + l, sl] * ps
                return carry

            lax.fori_loop(0, CH // 16, group, 0)

        bufs = [(inA, outA, pA, gsA, ssA, usA),
                (inB, outB, pB, gsB, ssB, usB)]

        def pair_body(c2, carry):
            for b, (inb, outb, pbuf, gs, ss, us) in enumerate(bufs):
                ci = 2 * c2 + b
                g_desc(ci, inb, gs).wait()

                @pl.when(c2 > 0)
                def _():
                    s_desc(ci - 2, outb, ss).wait()
                    u_desc(ci - 2, pbuf, us).wait()

                scale(ci, inb, outb, pbuf)
                g_desc(ci + 2, inb, gs).start()
                s_desc(ci, outb, ss).start(add=True)
                u_desc(ci, pbuf, us).start(add=True)
            return carry

        lax.fori_loop(0, nch // 2, pair_body, 0)

        s_desc(nch - 2, outA, ssA).wait()
        u_desc(nch - 2, pA, usA).wait()
        s_desc(nch - 1, outB, ssB).wait()
        u_desc(nch - 1, pB, usB).wait()
        g_desc(nch, inA, gsA).wait()
        g_desc(nch + 1, inB, gsB).wait()

        plsc.subcore_barrier()

        pltpu.sync_copy(acc_s.at[pl.ds(row0, rows_per_sub)],
                        acc_hbm.at[cid, sid])
        pltpu.sync_copy(sums_s.at[pl.ds(srow0, srows_per_sub)],
                        st_v.at[pl.ds(0, srows_per_sub)])
        pltpu.sync_copy(st_v.at[pl.ds(0, srows_per_sub)],
                        sums_hbm.at[pl.ds(cid * nsum + srow0, srows_per_sub)])

    return sc_kernel


def kernel(node_features, edges, W_hidden, b_hidden, W_att, b_att):
    n, d = node_features.shape
    h = W_hidden.shape[1]
    e = edges.shape[0]

    npad = ((n + 1 + BLK - 1) // BLK) * BLK
    nchunks = (e + NSUB * CH - 1) // (NSUB * CH)
    nch = nchunks + (nchunks & 1)
    epw = nch * CH
    epad = epw * NSUB

    xp = jnp.pad(node_features, ((0, npad - n), (0, 0)))
    wa = W_att.reshape(2, h).T
    bh2 = b_hidden.reshape(1, h)
    ba2 = jnp.concatenate([b_att, jnp.zeros((1,), jnp.float32)]).reshape(1, 2)
    pad_e = epad - e
    src_p = jnp.concatenate(
        [edges[:, 0], jnp.full((pad_e,), n, jnp.int32)]).reshape(NSUB, nch, CH)
    dst_p = jnp.concatenate(
        [edges[:, 1], jnp.full((pad_e,), n, jnp.int32)]).reshape(NSUB, nch, CH)
    padc = jnp.full((NSUB, 2, CH), n, jnp.int32)
    src_p = jnp.concatenate([src_p, padc], axis=1)
    dst_p = jnp.concatenate([dst_p, padc], axis=1)

    feat, ac = pl.pallas_call(
        _tc_feat_body,
        grid=(npad // BLK,),
        in_specs=[
            pl.BlockSpec((BLK, d), lambda i: (i, 0)),
            pl.BlockSpec((d, h), lambda i: (0, 0)),
            pl.BlockSpec((h, 2), lambda i: (0, 0)),
            pl.BlockSpec((1, h), lambda i: (0, 0)),
            pl.BlockSpec((1, 2), lambda i: (0, 0)),
        ],
        out_specs=[
            pl.BlockSpec((BLK, h), lambda i: (i, 0)),
            pl.BlockSpec((BLK, 2), lambda i: (i, 0)),
        ],
        out_shape=[
            jax.ShapeDtypeStruct((npad, h), jnp.float32),
            jax.ShapeDtypeStruct((npad, 2), jnp.float32),
        ],
    )(xp, W_hidden, wa, bh2, ba2)

    ntab = ((n + 1 + 7) // 8) * 8
    nsum = ((n + 1 + 127) // 128) * 128
    hh = h // 2
    feat2 = feat.reshape(2 * npad, hh)
    acc, sums = _make_sc_kernel(npad, nsum, ntab, nch, hh)(
        feat2, ac[:ntab, 0], ac[:ntab, 1], src_p, dst_p)
    acc = acc.reshape(NCORES, npad, hh)
    s0 = sums.reshape(NCORES, nsum)[0, :npad, None]

    out = pl.pallas_call(
        _tc_combine_body,
        grid=(npad // BLK,),
        in_specs=[
            pl.BlockSpec((BLK, hh), lambda i: (i, 0)),
            pl.BlockSpec((BLK, hh), lambda i: (i, 0)),
            pl.BlockSpec((BLK, 1), lambda i: (i, 0)),
        ],
        out_specs=pl.BlockSpec((BLK, h), lambda i: (i, 0)),
        out_shape=jax.ShapeDtypeStruct((npad, h), jnp.float32),
    )(acc[0], acc[1], s0)

    return out[:n]
